# baseline (speedup 1.0000x reference)
import functools

import jax
import jax.numpy as jnp
import numpy as np
from jax import lax
from jax.experimental import pallas as pl
from jax.experimental.pallas import tpu as pltpu
from jax.experimental.pallas import tpu_sc as plsc

B, N = 64, 4096
BEV = 200
PR, PC = 12, 16
HP, WP = BEV + 2 * PR, BEV + 2 * PC
SH, SW = HP + 1, 256

def _prefix_mat(n_out, n_in, pad):
    m = np.zeros((n_out, n_in), np.float32)
    src = np.clip(np.arange(n_out - 1) - pad, 0, n_in - 1)
    for k, j in enumerate(src):
        m[k + 1 :, j] += 1.0
    return m

M_R = _prefix_mat(SH, BEV, PR)
M_C = _prefix_mat(SW, BEV, PC).T.copy()


def _sat_body(mr_ref, mc_ref, inst_ref, driv_ref, s_inst_ref, s_head_ref,
              s_rule_ref):
    mr = mr_ref[...].astype(jnp.bfloat16)
    mc = mc_ref[...]
    gi = inst_ref[0]
    gd = driv_ref[0]
    g3 = jnp.concatenate(
        [gi, gi * gd, (gd == 0.0).astype(jnp.float32)], axis=1
    ).astype(jnp.bfloat16)
    t3 = jnp.dot(mr, g3, preferred_element_type=jnp.float32)
    tr = jnp.concatenate(
        [t3[:, :BEV], t3[:, BEV : 2 * BEV], t3[:, 2 * BEV :]], axis=0
    )
    tr_hi = tr.astype(jnp.bfloat16)
    tr_lo = (tr - tr_hi.astype(jnp.float32)).astype(jnp.bfloat16)
    mcb = mc.astype(jnp.bfloat16)
    s = (jnp.dot(tr_hi, mcb, preferred_element_type=jnp.float32)
         + jnp.dot(tr_lo, mcb, preferred_element_type=jnp.float32))
    s_inst_ref[0] = s[:SH]
    s_head_ref[0] = s[SH : 2 * SH]
    s_rule_ref[0] = s[2 * SH :]


def _build_sats(inst, driv):
    grid = (inst.shape[0],)
    full2 = pl.BlockSpec((SH, BEV), lambda b: (0, 0))
    full2c = pl.BlockSpec((BEV, SW), lambda b: (0, 0))
    per_b = pl.BlockSpec((1, BEV, BEV), lambda b: (b, 0, 0))
    out_b = pl.BlockSpec((1, SH, SW), lambda b: (b, 0, 0))
    out_shape = [jax.ShapeDtypeStruct((inst.shape[0], SH, SW),
                                      jnp.float32)] * 3
    return pl.pallas_call(
        _sat_body,
        grid=grid,
        in_specs=[full2, full2c, per_b, per_b],
        out_specs=[out_b, out_b, out_b],
        out_shape=out_shape,
    )(jnp.asarray(M_R), jnp.asarray(M_C), inst, driv)


def _prep_body(x_ref, y_ref, base_ref, baseh_ref, cva_ref, ego_ref):
    x = x_ref[...]
    y = y_ref[...]
    fx = jnp.floor(x * 2.0).astype(jnp.int32)
    fy = jnp.floor(y * 2.0).astype(jnp.int32)
    rb = jnp.clip(fx, -106, 106) + PR
    base_ref[...] = rb * SW + (jnp.clip(fy, -109, 107) + PC)
    baseh_ref[...] = rb * SW + (jnp.clip(fy + 20, -109, 107) + PC)
    cvr = jnp.clip(((x + 49.75) * 2.0).astype(jnp.int32), 0, BEV - 1)
    cvc = jnp.clip(((y + 49.75) * 2.0).astype(jnp.int32), 0, BEV - 1)
    cva_ref[...] = cvr * SW + cvc
    ego_ref[...] = jnp.sqrt(x * x + y * y) * 2.0


def _prep_points(x, y):
    rows = 8
    nb = x.shape[0]
    grid = (nb // rows,)
    spec = pl.BlockSpec((rows, N), lambda i: (i, 0))
    shapes = ([jax.ShapeDtypeStruct((nb, N), jnp.int32)] * 3
              + [jax.ShapeDtypeStruct((nb, N), jnp.float32)])
    return pl.pallas_call(
        _prep_body,
        grid=grid,
        in_specs=[spec, spec],
        out_specs=[spec] * 4,
        out_shape=shapes,
    )(x, y)


def _rect16(tab, r0, c0, dr, dc):
    p00 = plsc.load_gather(tab, [r0, c0])
    p01 = plsc.load_gather(tab, [r0, c0 + dc])
    p10 = plsc.load_gather(tab, [r0 + dr, c0])
    p11 = plsc.load_gather(tab, [r0 + dr, c0 + dc])
    return (p11 - p01) - (p10 - p00)


def _clip100(v):
    return jnp.minimum(jnp.maximum(v, 0.0), 100.0)


def _sc_body(bpt, s_inst, s_head, s_rule, cv, base_h, baseh_h, cva_h, ego_h,
             out_h, tab, cvv, base_v, baseh_v, cva_v, ego_v, acc_v, sem_c):
    wid = lax.axis_index("s") * 2 + lax.axis_index("c")

    for bloc in range(bpt):
        b = wid * bpt + bloc
        cv_dma = pltpu.make_async_copy(cv.at[b], cvv, sem_c)
        cv_dma.start()
        pltpu.sync_copy(base_h.at[b], base_v)
        pltpu.sync_copy(baseh_h.at[b], baseh_v)
        pltpu.sync_copy(cva_h.at[b], cva_v)
        pltpu.sync_copy(ego_h.at[b], ego_v)

        pltpu.sync_copy(s_inst.at[b], tab)

        @pl.loop(0, N // 16)
        def _safety(i):
            o = pl.multiple_of(i * 16, 16)
            base = base_v[pl.ds(o, 16)]
            r = lax.shift_right_logical(base, 8)
            c = lax.bitwise_and(base, 255)
            eg = ego_v[pl.ds(o, 16)]
            s1 = _rect16(tab, r + 98, c + 97, 4, 8)
            s2 = _rect16(tab, r + 94, c + 93, 12, 16)
            acc_v[pl.ds(o, 16)] = _clip100((s1 + s2 * eg) * 0.1)

        pltpu.sync_copy(s_head.at[b], tab)

        @pl.loop(0, N // 16)
        def _headway(i):
            o = pl.multiple_of(i * 16, 16)
            bh = baseh_v[pl.ds(o, 16)]
            hw = _rect16(tab, lax.shift_right_logical(bh, 8) + 98,
                         lax.bitwise_and(bh, 255) + 97, 4, 8)
            acc_v[pl.ds(o, 16)] = acc_v[pl.ds(o, 16)] + _clip100(hw)

        pltpu.sync_copy(s_rule.at[b], tab)
        cv_dma.wait()

        @pl.loop(0, N // 16)
        def _rule_cv(i):
            o = pl.multiple_of(i * 16, 16)
            base = base_v[pl.ds(o, 16)]
            ru = _rect16(tab, lax.shift_right_logical(base, 8) + 98,
                         lax.bitwise_and(base, 255) + 97, 4, 8) * 5.0
            ca = cva_v[pl.ds(o, 16)]
            cvx = plsc.load_gather(cvv, [lax.shift_right_logical(ca, 8),
                                         lax.bitwise_and(ca, 255)])
            acc_v[pl.ds(o, 16)] = (acc_v[pl.ds(o, 16)] + _clip100(ru)
                                   + _clip100(cvx * 100.0))

        pltpu.sync_copy(acc_v, out_h.at[b])


def _sc_gather(s_inst, s_head, s_rule, cv, base, baseh, cva, ego):
    nb = s_inst.shape[0]
    mesh = plsc.VectorSubcoreMesh(core_axis_name="c", subcore_axis_name="s",
                                  num_cores=2, num_subcores=16)
    f = pl.kernel(
        functools.partial(_sc_body, nb // 32),
        out_type=jax.ShapeDtypeStruct((nb, N), jnp.float32),
        mesh=mesh,
        compiler_params=pltpu.CompilerParams(needs_layout_passes=False),
        scratch_types=[
            pltpu.VMEM((SH, SW), jnp.float32),
            pltpu.VMEM((BEV, BEV), jnp.float32),
            pltpu.VMEM((N,), jnp.int32),
            pltpu.VMEM((N,), jnp.int32),
            pltpu.VMEM((N,), jnp.int32),
            pltpu.VMEM((N,), jnp.float32),
            pltpu.VMEM((N,), jnp.float32),
            pltpu.SemaphoreType.DMA,
        ],
    )
    return f(s_inst, s_head, s_rule, cv, base, baseh, cva, ego)


def kernel(cost_volume, trajs, instance_occupancy, drivable_area):
    x = trajs[..., 0]
    y = trajs[..., 1]
    outs = []
    h = B // 2
    for lo in (0, h):
        sl = slice(lo, lo + h)
        s_inst, s_head, s_rule = _build_sats(instance_occupancy[sl],
                                             drivable_area[sl])
        base, baseh, cva, ego = _prep_points(x[sl], y[sl])
        outs.append(_sc_gather(s_inst, s_head, s_rule, cost_volume[sl],
                               base, baseh, cva, ego))
    return jnp.concatenate(outs, axis=0)


# speedup vs baseline: 1.0619x; 1.0619x over previous
import functools

import jax
import jax.numpy as jnp
import numpy as np
from jax import lax
from jax.experimental import pallas as pl
from jax.experimental.pallas import tpu as pltpu
from jax.experimental.pallas import tpu_sc as plsc

B, N = 64, 4096
BEV = 200
PR, PC = 12, 16
HP, WP = BEV + 2 * PR, BEV + 2 * PC
SH, SW = HP + 1, 256

def _prefix_mat(n_out, n_in, pad):
    m = np.zeros((n_out, n_in), np.float32)
    src = np.clip(np.arange(n_out - 1) - pad, 0, n_in - 1)
    for k, j in enumerate(src):
        m[k + 1 :, j] += 1.0
    return m

M_R = _prefix_mat(SH, BEV, PR)
M_C = _prefix_mat(SW, BEV, PC).T.copy()


def _sat_body(mr_ref, mc_ref, inst_ref, driv_ref, s_inst_ref, s_head_ref,
              s_rule_ref):
    mr = mr_ref[...].astype(jnp.bfloat16)
    mc = mc_ref[...]
    gi = inst_ref[0]
    gd = driv_ref[0]
    g3 = jnp.concatenate(
        [gi, gi * gd, (gd == 0.0).astype(jnp.float32)], axis=1
    ).astype(jnp.bfloat16)
    t3 = jnp.dot(mr, g3, preferred_element_type=jnp.float32)
    tr = jnp.concatenate(
        [t3[:, :BEV], t3[:, BEV : 2 * BEV], t3[:, 2 * BEV :]], axis=0
    )
    tr_hi = tr.astype(jnp.bfloat16)
    tr_lo = (tr - tr_hi.astype(jnp.float32)).astype(jnp.bfloat16)
    mcb = mc.astype(jnp.bfloat16)
    s = (jnp.dot(tr_hi, mcb, preferred_element_type=jnp.float32)
         + jnp.dot(tr_lo, mcb, preferred_element_type=jnp.float32))
    s_inst_ref[0] = s[:SH]
    s_head_ref[0] = s[SH : 2 * SH]
    s_rule_ref[0] = s[2 * SH :]


def _build_sats(inst, driv):
    grid = (inst.shape[0],)
    full2 = pl.BlockSpec((SH, BEV), lambda b: (0, 0))
    full2c = pl.BlockSpec((BEV, SW), lambda b: (0, 0))
    per_b = pl.BlockSpec((1, BEV, BEV), lambda b: (b, 0, 0))
    out_b = pl.BlockSpec((1, SH, SW), lambda b: (b, 0, 0))
    out_shape = [jax.ShapeDtypeStruct((inst.shape[0], SH, SW),
                                      jnp.float32)] * 3
    return pl.pallas_call(
        _sat_body,
        grid=grid,
        in_specs=[full2, full2c, per_b, per_b],
        out_specs=[out_b, out_b, out_b],
        out_shape=out_shape,
    )(jnp.asarray(M_R), jnp.asarray(M_C), inst, driv)


def _prep_body(x_ref, y_ref, base_ref, baseh_ref, cva_ref, ego_ref):
    x = x_ref[...]
    y = y_ref[...]
    fx = jnp.floor(x * 2.0).astype(jnp.int32)
    fy = jnp.floor(y * 2.0).astype(jnp.int32)
    rb = jnp.clip(fx, -106, 106) + PR
    base_ref[...] = rb * SW + (jnp.clip(fy, -109, 107) + PC)
    baseh_ref[...] = rb * SW + (jnp.clip(fy + 20, -109, 107) + PC)
    cvr = jnp.clip(((x + 49.75) * 2.0).astype(jnp.int32), 0, BEV - 1)
    cvc = jnp.clip(((y + 49.75) * 2.0).astype(jnp.int32), 0, BEV - 1)
    cva_ref[...] = cvr * SW + cvc
    ego_ref[...] = jnp.sqrt(x * x + y * y) * 2.0


def _prep_points(x, y):
    rows = 8
    nb = x.shape[0]
    grid = (nb // rows,)
    spec = pl.BlockSpec((rows, N), lambda i: (i, 0))
    shapes = ([jax.ShapeDtypeStruct((nb, N), jnp.int32)] * 3
              + [jax.ShapeDtypeStruct((nb, N), jnp.float32)])
    return pl.pallas_call(
        _prep_body,
        grid=grid,
        in_specs=[spec, spec],
        out_specs=[spec] * 4,
        out_shape=shapes,
    )(x, y)


def _rect16(tab, r0, c0, dr, dc):
    p00 = plsc.load_gather(tab, [r0, c0])
    p01 = plsc.load_gather(tab, [r0, c0 + dc])
    p10 = plsc.load_gather(tab, [r0 + dr, c0])
    p11 = plsc.load_gather(tab, [r0 + dr, c0 + dc])
    return (p11 - p01) - (p10 - p00)


def _clip100(v):
    return jnp.minimum(jnp.maximum(v, 0.0), 100.0)


def _sc_body(bpt, s_inst, s_head, s_rule, cv, base_h, baseh_h, cva_h, ego_h,
             out_h, tab, cvv, base_v, baseh_v, cva_v, ego_v, acc_v, sem_c):
    wid = lax.axis_index("s") * 2 + lax.axis_index("c")

    for bloc in range(bpt):
        b = wid * bpt + bloc
        cv_dma = pltpu.make_async_copy(cv.at[b], cvv, sem_c)
        cv_dma.start()
        pltpu.sync_copy(base_h.at[b], base_v)
        pltpu.sync_copy(baseh_h.at[b], baseh_v)
        pltpu.sync_copy(cva_h.at[b], cva_v)
        pltpu.sync_copy(ego_h.at[b], ego_v)

        pltpu.sync_copy(s_inst.at[b], tab)

        @pl.loop(0, N // 16)
        def _safety(i):
            o = pl.multiple_of(i * 16, 16)
            base = base_v[pl.ds(o, 16)]
            r = lax.shift_right_logical(base, 8)
            c = lax.bitwise_and(base, 255)
            eg = ego_v[pl.ds(o, 16)]
            s1 = _rect16(tab, r + 98, c + 97, 4, 8)
            s2 = _rect16(tab, r + 94, c + 93, 12, 16)
            acc_v[pl.ds(o, 16)] = _clip100((s1 + s2 * eg) * 0.1)

        pltpu.sync_copy(s_head.at[b], tab)

        @pl.loop(0, N // 16)
        def _headway(i):
            o = pl.multiple_of(i * 16, 16)
            bh = baseh_v[pl.ds(o, 16)]
            hw = _rect16(tab, lax.shift_right_logical(bh, 8) + 98,
                         lax.bitwise_and(bh, 255) + 97, 4, 8)
            acc_v[pl.ds(o, 16)] = acc_v[pl.ds(o, 16)] + _clip100(hw)

        pltpu.sync_copy(s_rule.at[b], tab)
        cv_dma.wait()

        @pl.loop(0, N // 16)
        def _rule_cv(i):
            o = pl.multiple_of(i * 16, 16)
            base = base_v[pl.ds(o, 16)]
            ru = _rect16(tab, lax.shift_right_logical(base, 8) + 98,
                         lax.bitwise_and(base, 255) + 97, 4, 8) * 5.0
            ca = cva_v[pl.ds(o, 16)]
            cvx = plsc.load_gather(cvv, [lax.shift_right_logical(ca, 8),
                                         lax.bitwise_and(ca, 255)])
            acc_v[pl.ds(o, 16)] = (acc_v[pl.ds(o, 16)] + _clip100(ru)
                                   + _clip100(cvx * 100.0))

        pltpu.sync_copy(acc_v, out_h.at[b])


def _sc_gather(s_inst, s_head, s_rule, cv, base, baseh, cva, ego):
    nb = s_inst.shape[0]
    mesh = plsc.VectorSubcoreMesh(core_axis_name="c", subcore_axis_name="s",
                                  num_cores=2, num_subcores=16)
    f = pl.kernel(
        functools.partial(_sc_body, nb // 32),
        out_type=jax.ShapeDtypeStruct((nb, N), jnp.float32),
        mesh=mesh,
        compiler_params=pltpu.CompilerParams(needs_layout_passes=False),
        scratch_types=[
            pltpu.VMEM((SH, SW), jnp.float32),
            pltpu.VMEM((BEV, BEV), jnp.float32),
            pltpu.VMEM((N,), jnp.int32),
            pltpu.VMEM((N,), jnp.int32),
            pltpu.VMEM((N,), jnp.int32),
            pltpu.VMEM((N,), jnp.float32),
            pltpu.VMEM((N,), jnp.float32),
            pltpu.SemaphoreType.DMA,
        ],
    )
    return f(s_inst, s_head, s_rule, cv, base, baseh, cva, ego)


def kernel(cost_volume, trajs, instance_occupancy, drivable_area):
    x = trajs[..., 0]
    y = trajs[..., 1]
    s_inst, s_head, s_rule = _build_sats(instance_occupancy, drivable_area)
    base, baseh, cva, ego = _prep_points(x, y)
    return _sc_gather(s_inst, s_head, s_rule, cost_volume,
                      base, baseh, cva, ego)
